# baseline (speedup 1.0000x reference)
import functools

import jax
import jax.numpy as jnp
from jax import lax
from jax.experimental import pallas as pl
from jax.experimental.pallas import tpu as pltpu
from jax.experimental.pallas import tpu_sc as plsc

B = 512
L = 100
NNODES = 1000
LOOKBACK = 30
NPAD = 1024
LPAD = 128
NC = 2
NS = 16
LANES = 16
NWORK = NC * NS

NCHUNK = 4
BCH = B // NCHUNK


def _proj_body(x_ref, w_ref, y_ref, z_ref):
    xb = x_ref[...]
    w = w_ref[...]
    wl = w[:, 0].reshape(1, LOOKBACK, 1)
    wr = w[:, 1].reshape(1, LOOKBACK, 1)
    y = jnp.sum(xb * wl, axis=1)
    z = jnp.sum(xb * wr, axis=1)
    pad = jnp.zeros((y.shape[0], NPAD - NNODES), jnp.float32)
    y_ref[...] = jnp.concatenate([y, pad], axis=1)
    z_ref[...] = jnp.concatenate([z, pad], axis=1)


def _project(x, w):
    nb = x.shape[0]
    Bb = 32
    return pl.pallas_call(
        _proj_body,
        grid=(nb // Bb,),
        in_specs=[
            pl.BlockSpec((Bb, LOOKBACK, NNODES), lambda b: (b, 0, 0)),
            pl.BlockSpec((LOOKBACK, 2), lambda b: (0, 0)),
        ],
        out_specs=[
            pl.BlockSpec((Bb, NPAD), lambda b: (b, 0)),
            pl.BlockSpec((Bb, NPAD), lambda b: (b, 0)),
        ],
        out_shape=[
            jax.ShapeDtypeStruct((nb, NPAD), jnp.float32),
            jax.ShapeDtypeStruct((nb, NPAD), jnp.float32),
        ],
    )(x, w)


def _seg_mean_body(spw, stops_hbm, y_hbm, agg_hbm, stops_v, y_v, sums_v, cnt_v, gv):
    cid = lax.axis_index("c")
    sid = lax.axis_index("s")
    wid = sid * NC + cid

    zero16f = jnp.zeros((LANES,), jnp.float32)
    ones16f = jnp.ones((LANES,), jnp.float32)
    one16f = jnp.full((LANES,), 1.0, jnp.float32)
    lane = lax.broadcasted_iota(jnp.int32, (LANES,), 0)
    tail_mask = lane < ((L - 1) - 6 * LANES)

    def body(j, carry):
        i = wid * spw + j
        pltpu.sync_copy(stops_hbm.at[i], stops_v)
        pltpu.sync_copy(y_hbm.at[i], y_v)
        for k in range(NPAD // LANES):
            sums_v[pl.ds(k * LANES, LANES)] = zero16f
            cnt_v[pl.ds(k * LANES, LANES)] = zero16f
        for c in range(7):
            idx = stops_v[pl.ds(c * LANES, LANES)]
            gv[pl.ds(c * LANES, LANES)] = plsc.load_gather(y_v, [idx])
        for c in range(7):
            didx = stops_v[pl.ds(c * LANES + 1, LANES)]
            vals = gv[pl.ds(c * LANES, LANES)]
            if c < 6:
                plsc.addupdate_scatter(sums_v, [didx], vals)
                plsc.addupdate_scatter(cnt_v, [didx], ones16f)
            else:
                plsc.addupdate_scatter(sums_v, [didx], vals, mask=tail_mask)
                plsc.addupdate_scatter(cnt_v, [didx], ones16f, mask=tail_mask)
        for k in range(NPAD // LANES):
            s = sums_v[pl.ds(k * LANES, LANES)]
            cc = cnt_v[pl.ds(k * LANES, LANES)]
            sums_v[pl.ds(k * LANES, LANES)] = s / jnp.maximum(cc, one16f)
        pltpu.sync_copy(sums_v, agg_hbm.at[i])
        return carry

    lax.fori_loop(0, spw, body, 0)


def _seg_mean(stops_padded, y):
    nb = y.shape[0]
    mesh = plsc.VectorSubcoreMesh(
        core_axis_name="c", subcore_axis_name="s", num_cores=NC, num_subcores=NS
    )
    f = pl.kernel(
        functools.partial(_seg_mean_body, nb // NWORK),
        out_type=jax.ShapeDtypeStruct((nb, NPAD), jnp.float32),
        mesh=mesh,
        scratch_types=[
            pltpu.VMEM((LPAD,), jnp.int32),
            pltpu.VMEM((NPAD,), jnp.float32),
            pltpu.VMEM((NPAD,), jnp.float32),
            pltpu.VMEM((NPAD,), jnp.float32),
            pltpu.VMEM((7 * LANES,), jnp.float32),
        ],
        compiler_params=pltpu.CompilerParams(needs_layout_passes=False),
    )
    return f(stops_padded, y)


def _logits_body(agg_ref, z_ref, dist_ref, mask_ref, wv_ref, out_ref):
    a = agg_ref[...][:, :NNODES]
    zz = z_ref[...][:, :NNODES]
    d = dist_ref[...]
    w_out = wv_ref[0]
    w_dist = wv_ref[1]
    logits = w_out * (a + zz) + w_dist * d
    m = jnp.max(logits, axis=1, keepdims=True)
    ex = jnp.exp(logits - m)
    lse = jnp.log(jnp.sum(ex, axis=1, keepdims=True)) + m
    logp = logits - lse
    msk = mask_ref[...] != 0
    out_ref[...] = jnp.where(msk, jnp.float32(-1e8), logp)


def _logits(agg, z, dist2d, x_mask, wv):
    nb = agg.shape[0]
    Bc = 64
    return pl.pallas_call(
        _logits_body,
        grid=(nb // Bc,),
        in_specs=[
            pl.BlockSpec((Bc, NPAD), lambda b: (b, 0)),
            pl.BlockSpec((Bc, NPAD), lambda b: (b, 0)),
            pl.BlockSpec((1, NNODES), lambda b: (0, 0)),
            pl.BlockSpec((Bc, NNODES), lambda b: (b, 0)),
            pl.BlockSpec(memory_space=pltpu.SMEM),
        ],
        out_specs=pl.BlockSpec((Bc, NNODES), lambda b: (b, 0)),
        out_shape=jax.ShapeDtypeStruct((nb, NNODES), jnp.float32),
    )(agg, z, dist2d, x_mask, wv)


def kernel(stops, x, x_dist, x_features, x_week, x_mask, stop_emb_table,
           week_emb_table, W_l, b_l, W_r, W_fc2, b_fc2):
    w = jnp.concatenate([W_l, W_r], axis=1)
    stops_padded = jnp.pad(stops, ((0, 0), (0, LPAD - L)))
    wv = jnp.stack([W_fc2[78, 0], W_fc2[79, 0]])
    dist2d = x_dist.reshape(1, NNODES)
    outs = []
    for c in range(NCHUNK):
        sl = slice(c * BCH, (c + 1) * BCH)
        y, z = _project(x[sl], w)
        agg = _seg_mean(stops_padded[sl], y)
        outs.append(_logits(agg, z, dist2d, x_mask[sl], wv))
    return jnp.concatenate(outs, axis=0)


# speedup vs baseline: 1.3255x; 1.3255x over previous
import functools

import jax
import jax.numpy as jnp
from jax import lax
from jax.experimental import pallas as pl
from jax.experimental.pallas import tpu as pltpu
from jax.experimental.pallas import tpu_sc as plsc

B = 512
L = 100
NNODES = 1000
LOOKBACK = 30
NPAD = 1024
LPAD = 128
NC = 2
NS = 16
LANES = 16
NWORK = NC * NS
SPW = B // NWORK


def _proj_body(x_ref, w_ref, y_ref, z_ref):
    xb = x_ref[...]
    w = w_ref[...]
    wl = w[:, 0].reshape(1, LOOKBACK, 1)
    wr = w[:, 1].reshape(1, LOOKBACK, 1)
    y = jnp.sum(xb * wl, axis=1)
    z = jnp.sum(xb * wr, axis=1)
    pad = jnp.zeros((y.shape[0], NPAD - NNODES), jnp.float32)
    y_ref[...] = jnp.concatenate([y, pad], axis=1)
    z_ref[...] = jnp.concatenate([z, pad], axis=1)


def _project(x, w):
    Bb = 32
    return pl.pallas_call(
        _proj_body,
        grid=(B // Bb,),
        in_specs=[
            pl.BlockSpec((Bb, LOOKBACK, NNODES), lambda b: (b, 0, 0)),
            pl.BlockSpec((LOOKBACK, 2), lambda b: (0, 0)),
        ],
        out_specs=[
            pl.BlockSpec((Bb, NPAD), lambda b: (b, 0)),
            pl.BlockSpec((Bb, NPAD), lambda b: (b, 0)),
        ],
        out_shape=[
            jax.ShapeDtypeStruct((B, NPAD), jnp.float32),
            jax.ShapeDtypeStruct((B, NPAD), jnp.float32),
        ],
    )(x, w)


def _seg_sample(stops_hbm, y_hbm, agg_hbm, stops_v, y_v, sums_v, cnt_v, gv,
                sem_out, i):
    zero16f = jnp.zeros((LANES,), jnp.float32)
    ones16f = jnp.ones((LANES,), jnp.float32)
    one16f = jnp.full((LANES,), 1.0, jnp.float32)
    lane = lax.broadcasted_iota(jnp.int32, (LANES,), 0)
    tail_mask = lane < ((L - 1) - 6 * LANES)

    for k in range(NPAD // LANES):
        sums_v[pl.ds(k * LANES, LANES)] = zero16f
        cnt_v[pl.ds(k * LANES, LANES)] = zero16f
    for c in range(7):
        idx = stops_v[pl.ds(c * LANES, LANES)]
        gv[pl.ds(c * LANES, LANES)] = plsc.load_gather(y_v, [idx])
    for c in range(7):
        didx = stops_v[pl.ds(c * LANES + 1, LANES)]
        vals = gv[pl.ds(c * LANES, LANES)]
        if c < 6:
            plsc.addupdate_scatter(sums_v, [didx], vals)
            plsc.addupdate_scatter(cnt_v, [didx], ones16f)
        else:
            plsc.addupdate_scatter(sums_v, [didx], vals, mask=tail_mask)
            plsc.addupdate_scatter(cnt_v, [didx], ones16f, mask=tail_mask)
    for k in range(NPAD // LANES):
        s = sums_v[pl.ds(k * LANES, LANES)]
        cc = cnt_v[pl.ds(k * LANES, LANES)]
        sums_v[pl.ds(k * LANES, LANES)] = s / jnp.maximum(cc, one16f)
    pltpu.async_copy(sums_v, agg_hbm.at[i], sem_out)


def _seg_mean_body(stops_hbm, y_hbm, agg_hbm, stops_v0, stops_v1, y_v0, y_v1,
                   sums_v0, sums_v1, cnt_v, gv, sem_in0, sem_in1,
                   sem_out0, sem_out1):
    cid = lax.axis_index("c")
    sid = lax.axis_index("s")
    wid = sid * NC + cid
    base = wid * SPW

    def start_in(i, sv, yv, sem):
        pltpu.async_copy(stops_hbm.at[i], sv, sem)
        pltpu.async_copy(y_hbm.at[i], yv, sem)

    def wait_in(i, sv, yv, sem):
        pltpu.make_async_copy(stops_hbm.at[i], sv, sem).wait()
        pltpu.make_async_copy(y_hbm.at[i], yv, sem).wait()

    def wait_out(sums_v, sem):
        pltpu.make_async_copy(sums_v, agg_hbm.at[base], sem).wait()

    start_in(base, stops_v0, y_v0, sem_in0)

    def body(j2, carry):
        i0 = base + 2 * j2
        i1 = i0 + 1
        wait_in(i0, stops_v0, y_v0, sem_in0)
        start_in(i1, stops_v1, y_v1, sem_in1)

        @pl.when(j2 >= 1)
        def _():
            wait_out(sums_v0, sem_out0)

        _seg_sample(stops_hbm, y_hbm, agg_hbm, stops_v0, y_v0, sums_v0, cnt_v,
                    gv, sem_out0, i0)
        wait_in(i1, stops_v1, y_v1, sem_in1)

        @pl.when(2 * j2 + 2 < SPW)
        def _():
            start_in(i0 + 2, stops_v0, y_v0, sem_in0)

        @pl.when(j2 >= 1)
        def _():
            wait_out(sums_v1, sem_out1)

        _seg_sample(stops_hbm, y_hbm, agg_hbm, stops_v1, y_v1, sums_v1, cnt_v,
                    gv, sem_out1, i1)
        return carry

    lax.fori_loop(0, SPW // 2, body, 0)
    wait_out(sums_v0, sem_out0)
    wait_out(sums_v1, sem_out1)


def _seg_mean(stops_padded, y):
    mesh = plsc.VectorSubcoreMesh(
        core_axis_name="c", subcore_axis_name="s", num_cores=NC, num_subcores=NS
    )
    f = pl.kernel(
        _seg_mean_body,
        out_type=jax.ShapeDtypeStruct((B, NPAD), jnp.float32),
        mesh=mesh,
        scratch_types=[
            pltpu.VMEM((LPAD,), jnp.int32),
            pltpu.VMEM((LPAD,), jnp.int32),
            pltpu.VMEM((NPAD,), jnp.float32),
            pltpu.VMEM((NPAD,), jnp.float32),
            pltpu.VMEM((NPAD,), jnp.float32),
            pltpu.VMEM((NPAD,), jnp.float32),
            pltpu.VMEM((NPAD,), jnp.float32),
            pltpu.VMEM((7 * LANES,), jnp.float32),
            pltpu.SemaphoreType.DMA,
            pltpu.SemaphoreType.DMA,
            pltpu.SemaphoreType.DMA,
            pltpu.SemaphoreType.DMA,
        ],
        compiler_params=pltpu.CompilerParams(needs_layout_passes=False),
    )
    return f(stops_padded, y)


def _logits_body(agg_ref, z_ref, dist_ref, mask_ref, wv_ref, out_ref):
    a = agg_ref[...][:, :NNODES]
    zz = z_ref[...][:, :NNODES]
    d = dist_ref[...]
    w_out = wv_ref[0]
    w_dist = wv_ref[1]
    logits = w_out * (a + zz) + w_dist * d
    m = jnp.max(logits, axis=1, keepdims=True)
    ex = jnp.exp(logits - m)
    lse = jnp.log(jnp.sum(ex, axis=1, keepdims=True)) + m
    logp = logits - lse
    msk = mask_ref[...] != 0
    out_ref[...] = jnp.where(msk, jnp.float32(-1e8), logp)


def _logits(agg, z, dist2d, x_mask, wv):
    Bc = 64
    return pl.pallas_call(
        _logits_body,
        grid=(B // Bc,),
        in_specs=[
            pl.BlockSpec((Bc, NPAD), lambda b: (b, 0)),
            pl.BlockSpec((Bc, NPAD), lambda b: (b, 0)),
            pl.BlockSpec((1, NNODES), lambda b: (0, 0)),
            pl.BlockSpec((Bc, NNODES), lambda b: (b, 0)),
            pl.BlockSpec(memory_space=pltpu.SMEM),
        ],
        out_specs=pl.BlockSpec((Bc, NNODES), lambda b: (b, 0)),
        out_shape=jax.ShapeDtypeStruct((B, NNODES), jnp.float32),
    )(agg, z, dist2d, x_mask, wv)


def kernel(stops, x, x_dist, x_features, x_week, x_mask, stop_emb_table,
           week_emb_table, W_l, b_l, W_r, W_fc2, b_fc2):
    w = jnp.concatenate([W_l, W_r], axis=1)
    y, z = _project(x, w)
    stops_padded = jnp.pad(stops, ((0, 0), (0, LPAD - L)))
    agg = _seg_mean(stops_padded, y)
    wv = jnp.stack([W_fc2[78, 0], W_fc2[79, 0]])
    dist2d = x_dist.reshape(1, NNODES)
    return _logits(agg, z, dist2d, x_mask, wv)


# speedup vs baseline: 1.3627x; 1.0281x over previous
import functools

import jax
import jax.numpy as jnp
from jax import lax
from jax.experimental import pallas as pl
from jax.experimental.pallas import tpu as pltpu
from jax.experimental.pallas import tpu_sc as plsc

B = 512
L = 100
NNODES = 1000
LOOKBACK = 30
NPAD = 1024
LPAD = 128
NC = 2
NS = 16
LANES = 16
NWORK = NC * NS
SPW = B // NWORK


def _proj_body(x_ref, w_ref, y_ref, z_ref):
    xb = x_ref[...]
    w = w_ref[...]
    wl = w[:, 0].reshape(1, LOOKBACK, 1)
    wr = w[:, 1].reshape(1, LOOKBACK, 1)
    y = jnp.sum(xb * wl, axis=1)
    z = jnp.sum(xb * wr, axis=1)
    pad = jnp.zeros((y.shape[0], NPAD - NNODES), jnp.float32)
    y_ref[...] = jnp.concatenate([y, pad], axis=1)
    z_ref[...] = jnp.concatenate([z, pad], axis=1)


def _project(x, w):
    Bb = 64
    return pl.pallas_call(
        _proj_body,
        grid=(B // Bb,),
        in_specs=[
            pl.BlockSpec((Bb, LOOKBACK, NNODES), lambda b: (b, 0, 0)),
            pl.BlockSpec((LOOKBACK, 2), lambda b: (0, 0)),
        ],
        out_specs=[
            pl.BlockSpec((Bb, NPAD), lambda b: (b, 0)),
            pl.BlockSpec((Bb, NPAD), lambda b: (b, 0)),
        ],
        out_shape=[
            jax.ShapeDtypeStruct((B, NPAD), jnp.float32),
            jax.ShapeDtypeStruct((B, NPAD), jnp.float32),
        ],
    )(x, w)


def _seg_sample(stops_hbm, y_hbm, agg_hbm, stops_v, y_v, sums_v, cnt_v, gv,
                sem_out, i):
    zero16f = jnp.zeros((LANES,), jnp.float32)
    ones16f = jnp.ones((LANES,), jnp.float32)
    one16f = jnp.full((LANES,), 1.0, jnp.float32)
    lane = lax.broadcasted_iota(jnp.int32, (LANES,), 0)
    tail_mask = lane < ((L - 1) - 6 * LANES)

    for k in range(NPAD // LANES):
        sums_v[pl.ds(k * LANES, LANES)] = zero16f
        cnt_v[pl.ds(k * LANES, LANES)] = zero16f
    for c in range(7):
        idx = stops_v[pl.ds(c * LANES, LANES)]
        gv[pl.ds(c * LANES, LANES)] = plsc.load_gather(y_v, [idx])
    for c in range(7):
        didx = stops_v[pl.ds(c * LANES + 1, LANES)]
        vals = gv[pl.ds(c * LANES, LANES)]
        if c < 6:
            plsc.addupdate_scatter(sums_v, [didx], vals)
            plsc.addupdate_scatter(cnt_v, [didx], ones16f)
        else:
            plsc.addupdate_scatter(sums_v, [didx], vals, mask=tail_mask)
            plsc.addupdate_scatter(cnt_v, [didx], ones16f, mask=tail_mask)
    for k in range(NPAD // LANES):
        s = sums_v[pl.ds(k * LANES, LANES)]
        cc = cnt_v[pl.ds(k * LANES, LANES)]
        sums_v[pl.ds(k * LANES, LANES)] = s / jnp.maximum(cc, one16f)
    pltpu.async_copy(sums_v, agg_hbm.at[i], sem_out)


def _seg_mean_body(stops_hbm, y_hbm, agg_hbm, stops_v0, stops_v1, y_v0, y_v1,
                   sums_v0, sums_v1, cnt_v, gv, sem_in0, sem_in1,
                   sem_out0, sem_out1):
    cid = lax.axis_index("c")
    sid = lax.axis_index("s")
    wid = sid * NC + cid
    base = wid * SPW

    def start_in(i, sv, yv, sem):
        pltpu.async_copy(stops_hbm.at[i], sv, sem)
        pltpu.async_copy(y_hbm.at[i], yv, sem)

    def wait_in(i, sv, yv, sem):
        pltpu.make_async_copy(stops_hbm.at[i], sv, sem).wait()
        pltpu.make_async_copy(y_hbm.at[i], yv, sem).wait()

    def wait_out(sums_v, sem):
        pltpu.make_async_copy(sums_v, agg_hbm.at[base], sem).wait()

    start_in(base, stops_v0, y_v0, sem_in0)

    def body(j2, carry):
        i0 = base + 2 * j2
        i1 = i0 + 1
        wait_in(i0, stops_v0, y_v0, sem_in0)
        start_in(i1, stops_v1, y_v1, sem_in1)

        @pl.when(j2 >= 1)
        def _():
            wait_out(sums_v0, sem_out0)

        _seg_sample(stops_hbm, y_hbm, agg_hbm, stops_v0, y_v0, sums_v0, cnt_v,
                    gv, sem_out0, i0)
        wait_in(i1, stops_v1, y_v1, sem_in1)

        @pl.when(2 * j2 + 2 < SPW)
        def _():
            start_in(i0 + 2, stops_v0, y_v0, sem_in0)

        @pl.when(j2 >= 1)
        def _():
            wait_out(sums_v1, sem_out1)

        _seg_sample(stops_hbm, y_hbm, agg_hbm, stops_v1, y_v1, sums_v1, cnt_v,
                    gv, sem_out1, i1)
        return carry

    lax.fori_loop(0, SPW // 2, body, 0)
    wait_out(sums_v0, sem_out0)
    wait_out(sums_v1, sem_out1)


def _seg_mean(stops_padded, y):
    mesh = plsc.VectorSubcoreMesh(
        core_axis_name="c", subcore_axis_name="s", num_cores=NC, num_subcores=NS
    )
    f = pl.kernel(
        _seg_mean_body,
        out_type=jax.ShapeDtypeStruct((B, NPAD), jnp.float32),
        mesh=mesh,
        scratch_types=[
            pltpu.VMEM((LPAD,), jnp.int32),
            pltpu.VMEM((LPAD,), jnp.int32),
            pltpu.VMEM((NPAD,), jnp.float32),
            pltpu.VMEM((NPAD,), jnp.float32),
            pltpu.VMEM((NPAD,), jnp.float32),
            pltpu.VMEM((NPAD,), jnp.float32),
            pltpu.VMEM((NPAD,), jnp.float32),
            pltpu.VMEM((7 * LANES,), jnp.float32),
            pltpu.SemaphoreType.DMA,
            pltpu.SemaphoreType.DMA,
            pltpu.SemaphoreType.DMA,
            pltpu.SemaphoreType.DMA,
        ],
        compiler_params=pltpu.CompilerParams(needs_layout_passes=False),
    )
    return f(stops_padded, y)


def _logits_body(agg_ref, z_ref, dist_ref, mask_ref, wv_ref, out_ref):
    a = agg_ref[...][:, :NNODES]
    zz = z_ref[...][:, :NNODES]
    d = dist_ref[...]
    w_out = wv_ref[0]
    w_dist = wv_ref[1]
    logits = w_out * (a + zz) + w_dist * d
    m = jnp.max(logits, axis=1, keepdims=True)
    ex = jnp.exp(logits - m)
    lse = jnp.log(jnp.sum(ex, axis=1, keepdims=True)) + m
    logp = logits - lse
    msk = mask_ref[...] != 0
    out_ref[...] = jnp.where(msk, jnp.float32(-1e8), logp)


def _logits(agg, z, dist2d, x_mask, wv):
    Bc = 64
    return pl.pallas_call(
        _logits_body,
        grid=(B // Bc,),
        in_specs=[
            pl.BlockSpec((Bc, NPAD), lambda b: (b, 0)),
            pl.BlockSpec((Bc, NPAD), lambda b: (b, 0)),
            pl.BlockSpec((1, NNODES), lambda b: (0, 0)),
            pl.BlockSpec((Bc, NNODES), lambda b: (b, 0)),
            pl.BlockSpec(memory_space=pltpu.SMEM),
        ],
        out_specs=pl.BlockSpec((Bc, NNODES), lambda b: (b, 0)),
        out_shape=jax.ShapeDtypeStruct((B, NNODES), jnp.float32),
    )(agg, z, dist2d, x_mask, wv)


def kernel(stops, x, x_dist, x_features, x_week, x_mask, stop_emb_table,
           week_emb_table, W_l, b_l, W_r, W_fc2, b_fc2):
    w = jnp.concatenate([W_l, W_r], axis=1)
    y, z = _project(x, w)
    stops_padded = jnp.pad(stops, ((0, 0), (0, LPAD - L)))
    agg = _seg_mean(stops_padded, y)
    wv = jnp.stack([W_fc2[78, 0], W_fc2[79, 0]])
    dist2d = x_dist.reshape(1, NNODES)
    return _logits(agg, z, dist2d, x_mask, wv)
